# Initial kernel scaffold; baseline (speedup 1.0000x reference)
import jax
import jax.numpy as jnp
from jax.experimental import pallas as pl


def kernel(x, edge_attr, pos, w1, r1, b1, w2, r2, b2, w3, r3, b3, fc1_w, fc1_b, fc2_w, fc2_b, edge_index, batch):
    raise NotImplementedError("write your pallas kernel here")


# speedup vs baseline: 2.5742x; 2.5742x over previous
import dataclasses
import functools

import jax
import jax.numpy as jnp
from jax import lax
from jax.experimental import pallas as pl
from jax.experimental.pallas import tpu as pltpu
from jax.experimental.pallas import tpu_sc as plsc

_N = 10000
_E = 160000
_B = 128
_K = 5
_KT = 25

_NC = 2
_NS = 16
_NW = _NC * _NS
_L = 16

_CH = 128
_NCHUNK = 40
_EW = _CH * _NCHUNK
_EP = _EW * _NW
_NA = 10240
_RPS = _NA // _NS
_NPW = _NA // _NW
_PT = 520

_HIGH = lax.Precision.HIGHEST


def _mesh():
    return plsc.VectorSubcoreMesh(
        core_axis_name="c", subcore_axis_name="s",
        num_cores=_NC, num_subcores=_NS)


def _sc_params():
    return pltpu.CompilerParams(needs_layout_passes=False,
                                use_tc_tiling_on_sc=False)


def _prep_kernel(ea0, ea1, rowr):
    def body(a0, a1, rr, *outs):
        v0 = a0[...] * float(_K - 1)
        lo0 = jnp.clip(jnp.floor(v0), 0.0, float(_K - 2))
        f0 = v0 - lo0
        v1 = a1[...] * float(_K - 1)
        lo1 = jnp.clip(jnp.floor(v1), 0.0, float(_K - 2))
        f1 = v1 - lo1
        l0 = lo0.astype(jnp.int32)
        l1 = lo1.astype(jnp.int32)
        rbase = rr[...] * _KT
        bs = [(1.0 - f0) * (1.0 - f1), (1.0 - f0) * f1,
              f0 * (1.0 - f1), f0 * f1]
        wi = [l0 + _K * l1, l0 + _K * l1 + _K,
              l0 + 1 + _K * l1, l0 + 1 + _K * l1 + _K]
        for t in range(4):
            outs[t][...] = bs[t]
            outs[4 + t][...] = rbase + wi[t]
    shp = ea0.shape
    return pl.pallas_call(
        body,
        out_shape=[jax.ShapeDtypeStruct(shp, jnp.float32)] * 4
        + [jax.ShapeDtypeStruct(shp, jnp.int32)] * 4,
    )(ea0, ea1, rowr)


def _dense_prologue(h, wall, root, bias):
    n, cin = h.shape
    ktout = wall.shape[1]
    cout = root.shape[1]
    blk = 1000

    def body(h_ref, w_ref, r_ref, b_ref, y_ref, xr_ref):
        hb = h_ref[...]
        y_ref[...] = jnp.dot(hb, w_ref[...],
                             preferred_element_type=jnp.float32,
                             precision=_HIGH)
        xr_ref[...] = jnp.dot(hb, r_ref[...],
                              preferred_element_type=jnp.float32,
                              precision=_HIGH) + b_ref[...]

    return pl.pallas_call(
        body,
        grid=(n // blk,),
        in_specs=[pl.BlockSpec((blk, cin), lambda i: (i, 0)),
                  pl.BlockSpec((cin, ktout), lambda i: (0, 0)),
                  pl.BlockSpec((cin, cout), lambda i: (0, 0)),
                  pl.BlockSpec((1, cout), lambda i: (0, 0))],
        out_specs=[pl.BlockSpec((blk, ktout), lambda i: (i, 0)),
                   pl.BlockSpec((blk, cout), lambda i: (i, 0))],
        out_shape=[jax.ShapeDtypeStruct((n, ktout), jnp.float32),
                   jax.ShapeDtypeStruct((n, cout), jnp.float32)],
    )(h, wall, root, bias)


def _epilogue(aggp, degp, xr):
    n, cout = xr.shape
    blk = 1000

    def body(a_ref, d_ref, x_ref, h_ref):
        agg = a_ref[0] + a_ref[1]
        dg = d_ref[0][:, :1] + d_ref[1][:, :1]
        dg = jnp.clip(dg, 1.0, None)
        v = agg / dg + x_ref[...]
        h_ref[...] = jnp.where(v > 0, v, jnp.exp(v) - 1.0)

    return pl.pallas_call(
        body,
        grid=(n // blk,),
        in_specs=[pl.BlockSpec((_NC, blk, cout), lambda i: (0, i, 0)),
                  pl.BlockSpec((_NC, blk, _L), lambda i: (0, i, 0)),
                  pl.BlockSpec((blk, cout), lambda i: (i, 0))],
        out_specs=pl.BlockSpec((blk, cout), lambda i: (i, 0)),
        out_shape=jax.ShapeDtypeStruct((n, cout), jnp.float32),
    )(aggp, degp, xr)


def _cluster_kernel(px, py, bt):
    def body(x_r, y_r, b_r, o_r):
        ix = jnp.clip(jnp.floor(x_r[...] / 14.0), 0.0, 1.0).astype(jnp.int32)
        iy = jnp.clip(jnp.floor(y_r[...] / 14.0), 0.0, 1.0).astype(jnp.int32)
        o_r[...] = b_r[...] * 4 + iy * 2 + ix
    return pl.pallas_call(
        body,
        out_shape=jax.ShapeDtypeStruct(px.shape, jnp.int32),
    )(px, py, bt)


def _pool_reduce(parts):
    def body(p_ref, o_ref):
        m = jnp.max(p_ref[...], axis=0)[:512]
        o_ref[...] = jnp.where(jnp.isfinite(m), m, 0.0)
    return pl.pallas_call(
        body,
        out_shape=jax.ShapeDtypeStruct((512, 64), jnp.float32),
    )(parts)


def _mlp(pm, w1, b1, w2, b2):
    def body(p_ref, w1_ref, b1_ref, w2_ref, b2_ref, o_ref):
        g = jnp.dot(p_ref[...], w1_ref[...],
                    preferred_element_type=jnp.float32,
                    precision=_HIGH) + b1_ref[...]
        g = jnp.where(g > 0, g, jnp.exp(g) - 1.0)
        o = jnp.dot(g, w2_ref[...],
                    preferred_element_type=jnp.float32,
                    precision=_HIGH) + b2_ref[...]
        m = jnp.max(o, axis=1, keepdims=True)
        s = jnp.log(jnp.sum(jnp.exp(o - m), axis=1, keepdims=True))
        o_ref[...] = o - m - s
    return pl.pallas_call(
        body,
        out_shape=jax.ShapeDtypeStruct((_B, 10), jnp.float32),
    )(pm, w1, b1, w2, b2)


def _sc_spline_edge(ytab, ridx, bas, col2, cout, with_deg):
    nseg = cout // _L
    outs = [jax.ShapeDtypeStruct((_NC, _NA, cout), jnp.float32)]
    scratch = [pltpu.VMEM((4 * _CH,), jnp.int32),
               pltpu.VMEM((4 * _CH,), jnp.float32),
               pltpu.VMEM((1, _CH), jnp.int32),
               pltpu.VMEM((4 * _CH, cout), jnp.float32),
               pltpu.VMEM((_CH, cout), jnp.float32),
               pltpu.VMEM((_CH, cout), jnp.float32),
               pltpu.VMEM_SHARED((_NA, cout), jnp.float32)]
    if with_deg:
        outs.append(jax.ShapeDtypeStruct((_NC, _NA, _L), jnp.float32))
        scratch.append(pltpu.VMEM((_CH, _L), jnp.float32))
        scratch.append(pltpu.VMEM_SHARED((_NA, _L), jnp.float32))

    zslices = [(i * 128, 128) for i in range(_RPS // 128)]

    def body(y_h, ridx_h, bas_h, col_h, *rest):
        if with_deg:
            (agg_o, deg_o, idx_v, bas_v, col_v, taps_v, msg_v, zer_v,
             agg_sh, ones_v, deg_sh) = rest
        else:
            (agg_o, idx_v, bas_v, col_v, taps_v, msg_v, zer_v,
             agg_sh) = rest
        cid = lax.axis_index("c")
        sid = lax.axis_index("s")
        w = sid * _NC + cid
        z16 = jnp.zeros((_L,), jnp.float32)

        @pl.loop(0, _CH)
        def _(r):
            for c in range(nseg):
                zer_v[r, pl.ds(c * _L, _L)] = z16

        rb = sid * _RPS
        for off, sz in zslices:
            pltpu.sync_copy(zer_v.at[pl.ds(0, sz)],
                            agg_sh.at[pl.ds(rb + off, sz)])
        if with_deg:
            @pl.loop(0, _CH)
            def _(r):
                ones_v[r] = z16
            for off, sz in zslices:
                pltpu.sync_copy(ones_v.at[pl.ds(0, sz)],
                                deg_sh.at[pl.ds(rb + off, sz)])
            one16 = z16 + 1.0

            @pl.loop(0, _CH)
            def _(r):
                ones_v[r] = one16

        plsc.subcore_barrier()

        @pl.loop(0, _NCHUNK)
        def _(ch):
            e0 = w * _EW + ch * _CH
            o4 = e0 * 4
            pltpu.sync_copy(ridx_h.at[pl.ds(o4, 4 * _CH)], idx_v)
            pltpu.sync_copy(bas_h.at[pl.ds(o4, 4 * _CH)], bas_v)
            pltpu.sync_copy(col_h.at[pl.ds(w * _NCHUNK + ch, 1)], col_v)
            for j in range(4):
                pltpu.sync_copy(y_h.at[idx_v.at[pl.ds(j * _CH, _CH)]],
                                taps_v.at[pl.ds(j * _CH, _CH)])

            @pl.loop(0, _CH)
            def _(e):
                b4 = []
                for t in range(4):
                    bi = jnp.zeros((_L,), jnp.int32) + (e * 4 + t)
                    b4.append(plsc.load_gather(bas_v, [bi]))
                for c in range(nseg):
                    acc = b4[0] * taps_v[e * 4, pl.ds(c * _L, _L)]
                    for t in range(1, 4):
                        acc = acc + b4[t] * taps_v[e * 4 + t,
                                                   pl.ds(c * _L, _L)]
                    msg_v[e, pl.ds(c * _L, _L)] = acc

            pltpu.sync_copy(msg_v, agg_sh.at[col_v.at[0]], add=True)
            if with_deg:
                pltpu.sync_copy(ones_v, deg_sh.at[col_v.at[0]], add=True)

        plsc.subcore_barrier()
        pltpu.sync_copy(agg_sh.at[pl.ds(rb, _RPS)],
                        agg_o.at[cid, pl.ds(rb, _RPS)])
        if with_deg:
            pltpu.sync_copy(deg_sh.at[pl.ds(rb, _RPS)],
                            deg_o.at[cid, pl.ds(rb, _RPS)])

    k = pl.kernel(body, out_type=tuple(outs), mesh=_mesh(),
                  scratch_types=scratch, compiler_params=_sc_params())
    return k(ytab, ridx, bas, col2)


def _sc_pool(hpad, cl2):
    scratch = [pltpu.VMEM((_NPW, 64), jnp.float32),
               pltpu.VMEM((1, _NPW), jnp.int32),
               pltpu.VMEM((_PT, 64), jnp.float32)]

    def body(h_h, c_h, o_h, hbuf, cbuf, table):
        w = lax.axis_index("s") * _NC + lax.axis_index("c")
        ninf = jnp.zeros((_L,), jnp.float32) - jnp.inf

        @pl.loop(0, _PT)
        def _(r):
            for c in range(4):
                table[r, pl.ds(c * _L, _L)] = ninf

        pltpu.sync_copy(h_h.at[pl.ds(w * _NPW, _NPW)], hbuf)
        pltpu.sync_copy(c_h.at[pl.ds(w, 1)], cbuf)
        z = jnp.zeros((_L,), jnp.int32)
        lane = lax.iota(jnp.int32, 16)

        @pl.loop(0, _NPW)
        def _(i):
            cv = plsc.load_gather(cbuf, [z, z + i])
            for c in range(4):
                ci = lane + c * _L
                g = plsc.load_gather(table, [cv, ci])
                m = jnp.maximum(g, hbuf[i, pl.ds(c * _L, _L)])
                plsc.store_scatter(table, [cv, ci], m)

        pltpu.sync_copy(table, o_h.at[w])

    k = pl.kernel(body,
                  out_type=jax.ShapeDtypeStruct((_NW, _PT, 64), jnp.float32),
                  mesh=_mesh(), scratch_types=scratch,
                  compiler_params=_sc_params())
    return k(hpad, cl2)


def kernel(x, edge_attr, pos, w1, r1, b1, w2, r2, b2, w3, r3, b3,
           fc1_w, fc1_b, fc2_w, fc2_b, edge_index, batch):
    row = edge_index[0].astype(jnp.int32)
    col = edge_index[1].astype(jnp.int32)

    ea0 = edge_attr[:, 0].reshape(1250, 128)
    ea1 = edge_attr[:, 1].reshape(1250, 128)
    rowr = row.reshape(1250, 128)
    prep = _prep_kernel(ea0, ea1, rowr)
    bas = jnp.stack([t.reshape(-1) for t in prep[:4]], 1).reshape(-1)
    rid = jnp.stack([t.reshape(-1) for t in prep[4:]], 1).reshape(-1)
    bas = jnp.pad(bas, (0, 4 * (_EP - _E)))
    rid = jnp.pad(rid, (0, 4 * (_EP - _E)))
    col2 = jnp.pad(col, (0, _EP - _E),
                   constant_values=_N).reshape(_EP // _CH, _CH)

    xp = jnp.pad(x, ((0, 0), (0, 7)))
    w1a = jnp.pad(w1.transpose(1, 0, 2).reshape(1, _KT * 32),
                  ((0, 7), (0, 0)))
    r1p = jnp.pad(r1, ((0, 7), (0, 0)))
    y1, xr1 = _dense_prologue(xp, w1a, r1p, b1.reshape(1, -1))
    agg1, degp = _sc_spline_edge(y1.reshape(_N * _KT, 32), rid, bas, col2,
                                 32, True)
    h1 = _epilogue(agg1, degp, xr1)

    w2a = w2.transpose(1, 0, 2).reshape(32, _KT * 64)
    y2, xr2 = _dense_prologue(h1, w2a, r2, b2.reshape(1, -1))
    (agg2,) = _sc_spline_edge(y2.reshape(_N * _KT, 64), rid, bas, col2,
                              64, False)
    h2 = _epilogue(agg2, degp, xr2)

    w3a = w3.transpose(1, 0, 2).reshape(64, _KT * 64)
    y3, xr3 = _dense_prologue(h2, w3a, r3, b3.reshape(1, -1))
    (agg3,) = _sc_spline_edge(y3.reshape(_N * _KT, 64), rid, bas, col2,
                              64, False)
    h3 = _epilogue(agg3, degp, xr3)

    cl = _cluster_kernel(pos[:, 0].reshape(80, 125),
                         pos[:, 1].reshape(80, 125),
                         batch.astype(jnp.int32).reshape(80, 125))
    clp = jnp.pad(cl.reshape(-1), (0, _NA - _N),
                  constant_values=512).reshape(_NW, _NPW)
    hp = jnp.pad(h3, ((0, _NA - _N), (0, 0)))
    parts = _sc_pool(hp, clp)
    pooled = _pool_reduce(parts)

    pm = pooled.reshape(_B, 256)
    return _mlp(pm, fc1_w, fc1_b.reshape(1, -1), fc2_w, fc2_b.reshape(1, -1))
